# baseline (speedup 1.0000x reference)
import functools

import jax
import jax.numpy as jnp
import numpy as np
from jax import lax
from jax.experimental import pallas as pl
from jax.experimental.pallas import tpu as pltpu
from jax.experimental.pallas import tpu_sc as plsc

G_GRAPHS = 64
NC = 2
NS = 16
NW = NC * NS
CHUNK = 128
ROWS_PER_TILE = 632
NPAD = NS * ROWS_PER_TILE

_F32 = jnp.float32


def _mesh():
    return plsc.VectorSubcoreMesh(core_axis_name="c", subcore_axis_name="s")


DEGW = 16


def _sc_degree(cols3, ones_col, zero_col):
    ch = cols3.shape[1]

    @functools.partial(
        pl.kernel,
        mesh=_mesh(),
        out_type=jax.ShapeDtypeStruct((NC, NPAD, DEGW), _F32),
        compiler_params=pltpu.CompilerParams(use_tc_tiling_on_sc=False),
        scratch_types=[
            pltpu.VMEM((ch, CHUNK), jnp.int32),
            pltpu.VMEM((CHUNK, DEGW), _F32),
            pltpu.VMEM_SHARED((NPAD, DEGW), _F32),
        ],
    )
    def deg_k(col_hbm, ones_hbm, zero_hbm, out_hbm, col_v, ones_v, deg_sh):
        c = lax.axis_index("c")
        s = lax.axis_index("s")
        w = c * NS + s
        pltpu.sync_copy(col_hbm.at[w], col_v)
        pltpu.sync_copy(ones_hbm, ones_v)
        pltpu.sync_copy(zero_hbm, deg_sh.at[pl.ds(s * ROWS_PER_TILE, ROWS_PER_TILE)])
        plsc.subcore_barrier()

        def body(j, carry):
            pltpu.sync_copy(ones_v, deg_sh.at[col_v.at[j]], add=True)
            return carry

        lax.fori_loop(0, ch, body, 0)
        plsc.subcore_barrier()
        pltpu.sync_copy(
            deg_sh.at[pl.ds(s * ROWS_PER_TILE, ROWS_PER_TILE)],
            out_hbm.at[c, pl.ds(s * ROWS_PER_TILE, ROWS_PER_TILE)],
        )

    return deg_k(cols3, ones_col, zero_col)


def _sc_aggregate(y, rows3, cols3, zero_blk):
    h = y.shape[1]
    ch = rows3.shape[1]

    @functools.partial(
        pl.kernel,
        mesh=_mesh(),
        out_type=jax.ShapeDtypeStruct((NC, NPAD, h), _F32),
        compiler_params=pltpu.CompilerParams(use_tc_tiling_on_sc=False),
        scratch_types=[
            pltpu.VMEM((ch, CHUNK), jnp.int32),
            pltpu.VMEM((ch, CHUNK), jnp.int32),
            pltpu.VMEM((4, CHUNK, h), _F32),
            pltpu.VMEM_SHARED((NPAD, h), _F32),
        ]
        + [pltpu.SemaphoreType.DMA] * 4,
    )
    def agg_k(y_hbm, row_hbm, col_hbm, zero_hbm, out_hbm,
              row_v, col_v, gbuf, z_sh, *sems):
        c = lax.axis_index("c")
        s = lax.axis_index("s")
        w = c * NS + s
        pltpu.sync_copy(row_hbm.at[w], row_v)
        pltpu.sync_copy(col_hbm.at[w], col_v)
        pltpu.sync_copy(zero_hbm, z_sh.at[pl.ds(s * ROWS_PER_TILE, ROWS_PER_TILE)])
        plsc.subcore_barrier()

        nb = 4
        for b in range(nb):
            pltpu.async_copy(y_hbm.at[row_v.at[b]], gbuf.at[b], sems[b])

        def body(i, carry):
            for b in range(nb):
                j = nb * i + b
                pltpu.make_async_copy(y_hbm.at[row_v.at[j]], gbuf.at[b], sems[b]).wait()
                pltpu.sync_copy(gbuf.at[b], z_sh.at[col_v.at[j]], add=True)

                @pl.when(j + nb < ch)
                def _():
                    pltpu.async_copy(y_hbm.at[row_v.at[j + nb]], gbuf.at[b], sems[b])

            return carry

        lax.fori_loop(0, ch // nb, body, 0)
        plsc.subcore_barrier()
        pltpu.sync_copy(
            z_sh.at[pl.ds(s * ROWS_PER_TILE, ROWS_PER_TILE)],
            out_hbm.at[c, pl.ds(s * ROWS_PER_TILE, ROWS_PER_TILE)],
        )

    return agg_k(y, rows3, cols3, zero_blk)


def _pair_expand(dis2):
    rj = lax.broadcasted_iota(jnp.int32, (2, 128), 0)
    rl = lax.broadcasted_iota(jnp.int32, (2, 128), 1)
    r = jnp.where((rl >= 64).astype(jnp.int32) == rj, 1.0, 0.0).astype(_F32)
    return jnp.dot(dis2, r, preferred_element_type=_F32)


def _tc_prelayer(d0e, d0o, d1e, d1o, x2, w1b):
    nrp = d0e.shape[0]
    nr = x2.shape[0]

    def body(d0e_ref, d0o_ref, d1e_ref, d1o_ref, x2_ref, w_ref, dis2_ref, y_ref):
        dis_e = lax.rsqrt(d0e_ref[...] + d1e_ref[...] + 1.0)
        dis_o = lax.rsqrt(d0o_ref[...] + d1o_ref[...] + 1.0)
        dis2 = jnp.concatenate([dis_e[0:nr], dis_o[0:nr]], axis=1)
        dis2_ref[...] = dis2
        xw = jnp.dot(x2_ref[...], w_ref[...], preferred_element_type=_F32)
        y_ref[...] = _pair_expand(dis2) * xw

    return pl.pallas_call(
        body,
        out_shape=(
            jax.ShapeDtypeStruct((nr, 2), _F32),
            jax.ShapeDtypeStruct((nr, 128), _F32),
        ),
    )(d0e, d0o, d1e, d1o, x2, w1b)


def _tc_midlayer(zv, y128, dis2, b128, w2b):
    nr = y128.shape[0]

    def body(z_ref, y_ref, dis2_ref, b_ref, w_ref, out_ref):
        dis128 = _pair_expand(dis2_ref[...])
        u = z_ref[0][0:nr] + z_ref[1][0:nr] + y_ref[...]
        hid = jax.nn.relu(dis128 * u + b_ref[...])
        out_ref[...] = dis128 * jnp.dot(hid, w_ref[...], preferred_element_type=_F32)

    return pl.pallas_call(
        body,
        out_shape=jax.ShapeDtypeStruct((nr, 128), _F32),
    )(zv, y128, dis2, b128, w2b)


def _tc_head(zv, y128, dis2, b128, batch_e, batch_o, wl, bl):
    nr = y128.shape[0]
    h = 64

    def body(z_ref, y_ref, dis2_ref, b_ref, be_ref, bo_ref, wl_ref, bl_ref, out_ref):
        dis128 = _pair_expand(dis2_ref[...])
        u = z_ref[0][0:nr] + z_ref[1][0:nr] + y_ref[...]
        hid = jax.nn.relu(dis128 * u + b_ref[...])
        gids = lax.broadcasted_iota(jnp.int32, (G_GRAPHS, nr), 0)
        mask_e = jnp.where(be_ref[...] == gids, 1.0, 0.0).astype(_F32)
        mask_o = jnp.where(bo_ref[...] == gids, 1.0, 0.0).astype(_F32)
        sums = (jnp.dot(mask_e, hid[:, 0:h], preferred_element_type=_F32)
                + jnp.dot(mask_o, hid[:, h:2 * h], preferred_element_type=_F32))
        cnt = (jnp.sum(mask_e, axis=1, keepdims=True)
               + jnp.sum(mask_o, axis=1, keepdims=True))
        pooled = sums / jnp.maximum(cnt, 1.0)
        logits = jnp.dot(pooled, wl_ref[...], preferred_element_type=_F32) + bl_ref[...]
        out_ref[...] = jax.nn.sigmoid(logits)

    return pl.pallas_call(
        body,
        out_shape=jax.ShapeDtypeStruct((G_GRAPHS, 1), _F32),
    )(zv, y128, dis2, b128, batch_e, batch_o, wl, bl)


def kernel(x, edge_index, batch, W1, b1, W2, b2, Wl, bl):
    n = x.shape[0]
    e = edge_index.shape[1]

    ch = -(-e // (NW * CHUNK))
    ch = -(-ch // 4) * 4
    epad = NW * ch * CHUNK
    npad_edges = epad - e
    pad_i = np.arange(npad_edges, dtype=np.int32)
    pad_rows = jnp.asarray(pad_i % n)
    pad_cols = jnp.asarray(n + pad_i % (NPAD - n))
    rows3 = jnp.concatenate([edge_index[0], pad_rows]).reshape(NW, ch, CHUNK)
    cols3 = jnp.concatenate([edge_index[1], pad_cols]).reshape(NW, ch, CHUNK)

    ones_col = jnp.ones((CHUNK, DEGW), _F32)
    zero_col = jnp.zeros((ROWS_PER_TILE, DEGW), _F32)
    zero_blk = jnp.zeros((ROWS_PER_TILE, W1.shape[1]), _F32)

    hh = W1.shape[1]
    dd = x.shape[1]

    w1b = jnp.zeros((2 * dd, 2 * hh), _F32)
    w1b = w1b.at[0:dd, 0:hh].set(W1).at[dd:2 * dd, hh:2 * hh].set(W1)
    w2b = jnp.zeros((2 * hh, 2 * hh), _F32)
    w2b = w2b.at[0:hh, 0:hh].set(W2).at[hh:2 * hh, hh:2 * hh].set(W2)
    b1p = jnp.concatenate([b1, b1]).reshape(1, 2 * hh)
    b2p = jnp.concatenate([b2, b2]).reshape(1, 2 * hh)
    bi = batch.astype(jnp.int32)
    batch_e = bi[0::2].reshape(1, n // 2)
    batch_o = bi[1::2].reshape(1, n // 2)
    x2 = x.reshape(n // 2, 2 * dd)

    degp = _sc_degree(cols3, ones_col, zero_col)
    d0e = degp[0, 0::2, 0:1]
    d0o = degp[0, 1::2, 0:1]
    d1e = degp[1, 0::2, 0:1]
    d1o = degp[1, 1::2, 0:1]
    dis2, y1 = _tc_prelayer(d0e, d0o, d1e, d1o, x2, w1b)
    z1 = _sc_aggregate(y1.reshape(n, hh), rows3, cols3, zero_blk)
    z1v = z1.reshape(NC, NPAD * hh // 128, 128)
    y2 = _tc_midlayer(z1v, y1, dis2, b1p, w2b)
    z2 = _sc_aggregate(y2.reshape(n, hh), rows3, cols3, zero_blk)
    z2v = z2.reshape(NC, NPAD * hh // 128, 128)
    return _tc_head(z2v, y2, dis2, b2p, batch_e, batch_o, Wl, bl.reshape(1, 1))


# speedup vs baseline: 1.1945x; 1.1945x over previous
import functools

import jax
import jax.numpy as jnp
import numpy as np
from jax import lax
from jax.experimental import pallas as pl
from jax.experimental.pallas import tpu as pltpu
from jax.experimental.pallas import tpu_sc as plsc

G_GRAPHS = 64
NC = 2
NS = 16
NW = NC * NS
CHUNK = 128
ROWS_PER_TILE = 632
NPAD = NS * ROWS_PER_TILE

_F32 = jnp.float32


def _mesh():
    return plsc.VectorSubcoreMesh(core_axis_name="c", subcore_axis_name="s")


def _stage_idx(src2, pad2, dst, w, ch, lr):
    @pl.when(w < NW - 1)
    def _():
        pltpu.sync_copy(src2.at[pl.ds(w * ch, ch)], dst)

    @pl.when(w == NW - 1)
    def _():
        pltpu.sync_copy(src2.at[pl.ds((NW - 1) * ch, lr)], dst.at[pl.ds(0, lr)])
        pltpu.sync_copy(pad2, dst.at[pl.ds(lr, ch - lr)])


def _sc_degree(ec2, padc, ones_blk, zero_blk, ch, lr):
    h = ones_blk.shape[1]

    @functools.partial(
        pl.kernel,
        mesh=_mesh(),
        out_type=jax.ShapeDtypeStruct((NC, NPAD, h), _F32),
        compiler_params=pltpu.CompilerParams(use_tc_tiling_on_sc=False),
        scratch_types=[
            pltpu.VMEM((ch, CHUNK), jnp.int32),
            pltpu.VMEM((CHUNK, h), _F32),
            pltpu.VMEM_SHARED((NPAD, h), _F32),
        ],
    )
    def deg_k(col_hbm, pad_hbm, ones_hbm, zero_hbm, out_hbm, col_v, ones_v, deg_sh):
        c = lax.axis_index("c")
        s = lax.axis_index("s")
        w = c * NS + s
        _stage_idx(col_hbm, pad_hbm, col_v, w, ch, lr)
        pltpu.sync_copy(ones_hbm, ones_v)
        pltpu.sync_copy(zero_hbm, deg_sh.at[pl.ds(s * ROWS_PER_TILE, ROWS_PER_TILE)])
        plsc.subcore_barrier()

        def body(j, carry):
            pltpu.sync_copy(ones_v, deg_sh.at[col_v.at[j]], add=True)
            return carry

        lax.fori_loop(0, ch, body, 0)
        plsc.subcore_barrier()
        pltpu.sync_copy(
            deg_sh.at[pl.ds(s * ROWS_PER_TILE, ROWS_PER_TILE)],
            out_hbm.at[c, pl.ds(s * ROWS_PER_TILE, ROWS_PER_TILE)],
        )

    return deg_k(ec2, padc, ones_blk, zero_blk)


def _sc_aggregate(y, er2, ec2, padr, padc, zero_blk, ch, lr):
    h = y.shape[1]

    @functools.partial(
        pl.kernel,
        mesh=_mesh(),
        out_type=jax.ShapeDtypeStruct((NC, NPAD, h), _F32),
        compiler_params=pltpu.CompilerParams(use_tc_tiling_on_sc=False),
        scratch_types=[
            pltpu.VMEM((ch, CHUNK), jnp.int32),
            pltpu.VMEM((ch, CHUNK), jnp.int32),
            pltpu.VMEM((4, CHUNK, h), _F32),
            pltpu.VMEM_SHARED((NPAD, h), _F32),
        ]
        + [pltpu.SemaphoreType.DMA] * 4,
    )
    def agg_k(y_hbm, row_hbm, col_hbm, padr_hbm, padc_hbm, zero_hbm, out_hbm,
              row_v, col_v, gbuf, z_sh, *sems):
        c = lax.axis_index("c")
        s = lax.axis_index("s")
        w = c * NS + s
        _stage_idx(row_hbm, padr_hbm, row_v, w, ch, lr)
        _stage_idx(col_hbm, padc_hbm, col_v, w, ch, lr)
        pltpu.sync_copy(zero_hbm, z_sh.at[pl.ds(s * ROWS_PER_TILE, ROWS_PER_TILE)])
        plsc.subcore_barrier()

        nb = 4
        for b in range(nb):
            pltpu.async_copy(y_hbm.at[row_v.at[b]], gbuf.at[b], sems[b])

        def body(i, carry):
            for b in range(nb):
                j = nb * i + b
                pltpu.make_async_copy(y_hbm.at[row_v.at[j]], gbuf.at[b], sems[b]).wait()
                pltpu.sync_copy(gbuf.at[b], z_sh.at[col_v.at[j]], add=True)

                @pl.when(j + nb < ch)
                def _():
                    pltpu.async_copy(y_hbm.at[row_v.at[j + nb]], gbuf.at[b], sems[b])

            return carry

        lax.fori_loop(0, ch // nb, body, 0)
        plsc.subcore_barrier()
        pltpu.sync_copy(
            z_sh.at[pl.ds(s * ROWS_PER_TILE, ROWS_PER_TILE)],
            out_hbm.at[c, pl.ds(s * ROWS_PER_TILE, ROWS_PER_TILE)],
        )

    return agg_k(y, er2, ec2, padr, padc, zero_blk)


def _tc_prelayer(degv, x2, w1b):
    nr = x2.shape[0]

    def body(deg_ref, x2_ref, w_ref, dis_ref, y_ref):
        dis128 = lax.rsqrt(deg_ref[0] + deg_ref[1] + 1.0)
        dis_ref[...] = dis128
        xw = jnp.dot(x2_ref[...], w_ref[...], preferred_element_type=_F32)
        y_ref[...] = dis128[0:nr] * xw

    return pl.pallas_call(
        body,
        out_shape=(
            jax.ShapeDtypeStruct((NPAD // 2, 128), _F32),
            jax.ShapeDtypeStruct((nr, 128), _F32),
        ),
    )(degv, x2, w1b)


def _tc_midlayer(zv, y128, dis128, b128, w2b):
    nr = y128.shape[0]

    def body(z_ref, y_ref, dis_ref, b_ref, w_ref, out_ref):
        dis_n = dis_ref[0:nr]
        u = z_ref[0][0:nr] + z_ref[1][0:nr] + y_ref[...]
        hid = jax.nn.relu(dis_n * u + b_ref[...])
        out_ref[...] = dis_n * jnp.dot(hid, w_ref[...], preferred_element_type=_F32)

    return pl.pallas_call(
        body,
        out_shape=jax.ShapeDtypeStruct((nr, 128), _F32),
    )(zv, y128, dis128, b128, w2b)


def _tc_head(zv, y128, dis128, b128, batch_e, batch_o, wl, bl):
    nr = y128.shape[0]
    h = 64

    def body(z_ref, y_ref, dis_ref, b_ref, be_ref, bo_ref, wl_ref, bl_ref, out_ref):
        dis_n = dis_ref[0:nr]
        u = z_ref[0][0:nr] + z_ref[1][0:nr] + y_ref[...]
        hid = jax.nn.relu(dis_n * u + b_ref[...])
        gids = lax.broadcasted_iota(jnp.int32, (G_GRAPHS, nr), 0)
        mask_e = jnp.where(be_ref[...] == gids, 1.0, 0.0).astype(_F32)
        mask_o = jnp.where(bo_ref[...] == gids, 1.0, 0.0).astype(_F32)
        sums = (jnp.dot(mask_e, hid[:, 0:h], preferred_element_type=_F32)
                + jnp.dot(mask_o, hid[:, h:2 * h], preferred_element_type=_F32))
        cnt = (jnp.sum(mask_e, axis=1, keepdims=True)
               + jnp.sum(mask_o, axis=1, keepdims=True))
        pooled = sums / jnp.maximum(cnt, 1.0)
        logits = jnp.dot(pooled, wl_ref[...], preferred_element_type=_F32) + bl_ref[...]
        out_ref[...] = jax.nn.sigmoid(logits)

    return pl.pallas_call(
        body,
        out_shape=jax.ShapeDtypeStruct((G_GRAPHS, 1), _F32),
    )(zv, y128, dis128, b128, batch_e, batch_o, wl, bl)


def kernel(x, edge_index, batch, W1, b1, W2, b2, Wl, bl):
    n = x.shape[0]
    e = edge_index.shape[1]
    hh = W1.shape[1]
    dd = x.shape[1]

    ech = e // CHUNK
    ch = -(-ech // NW)
    ch = -(-ch // 4) * 4
    lr = ech - (NW - 1) * ch
    pc = ch - lr
    pad_i = np.arange(pc * CHUNK, dtype=np.int32)
    padr = jnp.asarray((pad_i % n).reshape(pc, CHUNK))
    padc = jnp.asarray((n + pad_i % (NPAD - n)).reshape(pc, CHUNK))
    er2 = edge_index[0].reshape(ech, CHUNK)
    ec2 = edge_index[1].reshape(ech, CHUNK)

    ones_blk = jnp.ones((CHUNK, hh), _F32)
    zero_blk = jnp.zeros((ROWS_PER_TILE, hh), _F32)

    w1b = jnp.zeros((2 * dd, 2 * hh), _F32)
    w1b = w1b.at[0:dd, 0:hh].set(W1).at[dd:2 * dd, hh:2 * hh].set(W1)
    w2b = jnp.zeros((2 * hh, 2 * hh), _F32)
    w2b = w2b.at[0:hh, 0:hh].set(W2).at[hh:2 * hh, hh:2 * hh].set(W2)
    b1p = jnp.concatenate([b1, b1]).reshape(1, 2 * hh)
    b2p = jnp.concatenate([b2, b2]).reshape(1, 2 * hh)
    bi = batch.astype(jnp.int32)
    batch_e = bi[0::2].reshape(1, n // 2)
    batch_o = bi[1::2].reshape(1, n // 2)
    x2 = x.reshape(n // 2, 2 * dd)

    degp = _sc_degree(ec2, padc, ones_blk, zero_blk, ch, lr)
    degv = degp.reshape(NC, NPAD // 2, 128)
    dis128, y1 = _tc_prelayer(degv, x2, w1b)
    z1 = _sc_aggregate(y1.reshape(n, hh), er2, ec2, padr, padc, zero_blk, ch, lr)
    z1v = z1.reshape(NC, NPAD // 2, 128)
    y2 = _tc_midlayer(z1v, y1, dis128, b1p, w2b)
    z2 = _sc_aggregate(y2.reshape(n, hh), er2, ec2, padr, padc, zero_blk, ch, lr)
    z2v = z2.reshape(NC, NPAD // 2, 128)
    return _tc_head(z2v, y2, dis128, b2p, batch_e, batch_o, Wl, bl.reshape(1, 1))


# speedup vs baseline: 1.1946x; 1.0001x over previous
import functools

import jax
import jax.numpy as jnp
import numpy as np
from jax import lax
from jax.experimental import pallas as pl
from jax.experimental.pallas import tpu as pltpu
from jax.experimental.pallas import tpu_sc as plsc

G_GRAPHS = 64
NC = 2
NS = 16
NW = NC * NS
CHUNK = 128
ROWS_PER_TILE = 632
NPAD = NS * ROWS_PER_TILE

_F32 = jnp.float32


def _mesh():
    return plsc.VectorSubcoreMesh(core_axis_name="c", subcore_axis_name="s")


def _stage_idx(src2, pad2, dst, w, ch, lr):
    @pl.when(w < NW - 1)
    def _():
        pltpu.sync_copy(src2.at[pl.ds(w * ch, ch)], dst)

    @pl.when(w == NW - 1)
    def _():
        pltpu.sync_copy(src2.at[pl.ds((NW - 1) * ch, lr)], dst.at[pl.ds(0, lr)])
        pltpu.sync_copy(pad2, dst.at[pl.ds(lr, ch - lr)])


def _sc_degree(ec2, padc, ones_blk, zero_blk, ch, lr):
    h = ones_blk.shape[1]

    @functools.partial(
        pl.kernel,
        mesh=_mesh(),
        out_type=jax.ShapeDtypeStruct((NC, NPAD, h), _F32),
        compiler_params=pltpu.CompilerParams(use_tc_tiling_on_sc=False),
        scratch_types=[
            pltpu.VMEM((ch, CHUNK), jnp.int32),
            pltpu.VMEM((CHUNK, h), _F32),
            pltpu.VMEM_SHARED((NPAD, h), _F32),
        ],
    )
    def deg_k(col_hbm, pad_hbm, ones_hbm, zero_hbm, out_hbm, col_v, ones_v, deg_sh):
        c = lax.axis_index("c")
        s = lax.axis_index("s")
        w = c * NS + s
        _stage_idx(col_hbm, pad_hbm, col_v, w, ch, lr)
        pltpu.sync_copy(ones_hbm, ones_v)
        pltpu.sync_copy(zero_hbm, deg_sh.at[pl.ds(s * ROWS_PER_TILE, ROWS_PER_TILE)])
        plsc.subcore_barrier()

        def body(j, carry):
            pltpu.sync_copy(ones_v, deg_sh.at[col_v.at[j]], add=True)
            return carry

        lax.fori_loop(0, ch, body, 0)
        plsc.subcore_barrier()
        pltpu.sync_copy(
            deg_sh.at[pl.ds(s * ROWS_PER_TILE, ROWS_PER_TILE)],
            out_hbm.at[c, pl.ds(s * ROWS_PER_TILE, ROWS_PER_TILE)],
        )

    return deg_k(ec2, padc, ones_blk, zero_blk)


def _sc_aggregate(y, er2, ec2, padr, padc, zero_blk, ch, lr):
    h = y.shape[1]

    @functools.partial(
        pl.kernel,
        mesh=_mesh(),
        out_type=jax.ShapeDtypeStruct((NC, NPAD, h), _F32),
        compiler_params=pltpu.CompilerParams(use_tc_tiling_on_sc=False),
        scratch_types=[
            pltpu.VMEM((ch, CHUNK), jnp.int32),
            pltpu.VMEM((ch, CHUNK), jnp.int32),
            pltpu.VMEM((4, CHUNK, h), _F32),
            pltpu.VMEM_SHARED((NPAD, h), _F32),
        ]
        + [pltpu.SemaphoreType.DMA] * 4,
    )
    def agg_k(y_hbm, row_hbm, col_hbm, padr_hbm, padc_hbm, zero_hbm, out_hbm,
              row_v, col_v, gbuf, z_sh, *sems):
        c = lax.axis_index("c")
        s = lax.axis_index("s")
        w = c * NS + s
        _stage_idx(row_hbm, padr_hbm, row_v, w, ch, lr)
        _stage_idx(col_hbm, padc_hbm, col_v, w, ch, lr)
        pltpu.sync_copy(zero_hbm, z_sh.at[pl.ds(s * ROWS_PER_TILE, ROWS_PER_TILE)])
        plsc.subcore_barrier()

        nb = 4
        for b in range(nb):
            pltpu.async_copy(y_hbm.at[row_v.at[b]], gbuf.at[b], sems[b])

        def body(i, carry):
            for b in range(nb):
                j = nb * i + b
                pltpu.make_async_copy(y_hbm.at[row_v.at[j]], gbuf.at[b], sems[b]).wait()
                pltpu.sync_copy(gbuf.at[b], z_sh.at[col_v.at[j]], add=True)

                @pl.when(j + nb < ch)
                def _():
                    pltpu.async_copy(y_hbm.at[row_v.at[j + nb]], gbuf.at[b], sems[b])

            return carry

        lax.fori_loop(0, ch // nb, body, 0)
        plsc.subcore_barrier()
        pltpu.sync_copy(
            z_sh.at[pl.ds(s * ROWS_PER_TILE, ROWS_PER_TILE)],
            out_hbm.at[c, pl.ds(s * ROWS_PER_TILE, ROWS_PER_TILE)],
        )

    return agg_k(y, er2, ec2, padr, padc, zero_blk)


def _tc_xw(x2, w1b):
    nr = x2.shape[0]

    def body(x2_ref, w_ref, y_ref):
        y_ref[...] = jnp.dot(x2_ref[...], w_ref[...], preferred_element_type=_F32)

    return pl.pallas_call(
        body,
        out_shape=jax.ShapeDtypeStruct((nr, 128), _F32),
    )(x2, w1b)


def _tc_prelayer(degv, xw128):
    nr = xw128.shape[0]

    def body(deg_ref, xw_ref, dis_ref, y_ref):
        dis128 = lax.rsqrt(deg_ref[0] + deg_ref[1] + 1.0)
        dis_ref[...] = dis128
        y_ref[...] = dis128[0:nr] * xw_ref[...]

    return pl.pallas_call(
        body,
        out_shape=(
            jax.ShapeDtypeStruct((NPAD // 2, 128), _F32),
            jax.ShapeDtypeStruct((nr, 128), _F32),
        ),
    )(degv, xw128)


def _tc_midlayer(zv, y128, dis128, b128, w2b):
    nr = y128.shape[0]

    def body(z_ref, y_ref, dis_ref, b_ref, w_ref, out_ref):
        dis_n = dis_ref[0:nr]
        u = z_ref[0][0:nr] + z_ref[1][0:nr] + y_ref[...]
        hid = jax.nn.relu(dis_n * u + b_ref[...])
        out_ref[...] = dis_n * jnp.dot(hid, w_ref[...], preferred_element_type=_F32)

    return pl.pallas_call(
        body,
        out_shape=jax.ShapeDtypeStruct((nr, 128), _F32),
    )(zv, y128, dis128, b128, w2b)


def _tc_head(zv, y128, dis128, b128, batch_e, batch_o, wl, bl):
    nr = y128.shape[0]
    h = 64

    def body(z_ref, y_ref, dis_ref, b_ref, be_ref, bo_ref, wl_ref, bl_ref, out_ref):
        dis_n = dis_ref[0:nr]
        u = z_ref[0][0:nr] + z_ref[1][0:nr] + y_ref[...]
        hid = jax.nn.relu(dis_n * u + b_ref[...])
        gids = lax.broadcasted_iota(jnp.int32, (G_GRAPHS, nr), 0)
        mask_e = jnp.where(be_ref[...] == gids, 1.0, 0.0).astype(_F32)
        mask_o = jnp.where(bo_ref[...] == gids, 1.0, 0.0).astype(_F32)
        sums = (jnp.dot(mask_e, hid[:, 0:h], preferred_element_type=_F32)
                + jnp.dot(mask_o, hid[:, h:2 * h], preferred_element_type=_F32))
        cnt = (jnp.sum(mask_e, axis=1, keepdims=True)
               + jnp.sum(mask_o, axis=1, keepdims=True))
        pooled = sums / jnp.maximum(cnt, 1.0)
        logits = jnp.dot(pooled, wl_ref[...], preferred_element_type=_F32) + bl_ref[...]
        out_ref[...] = jax.nn.sigmoid(logits)

    return pl.pallas_call(
        body,
        out_shape=jax.ShapeDtypeStruct((G_GRAPHS, 1), _F32),
    )(zv, y128, dis128, b128, batch_e, batch_o, wl, bl)


def kernel(x, edge_index, batch, W1, b1, W2, b2, Wl, bl):
    n = x.shape[0]
    e = edge_index.shape[1]
    hh = W1.shape[1]
    dd = x.shape[1]

    ech = e // CHUNK
    ch = -(-ech // NW)
    ch = -(-ch // 4) * 4
    lr = ech - (NW - 1) * ch
    pc = ch - lr
    pad_i = np.arange(pc * CHUNK, dtype=np.int32)
    padr = jnp.asarray((pad_i % n).reshape(pc, CHUNK))
    padc = jnp.asarray((n + pad_i % (NPAD - n)).reshape(pc, CHUNK))
    er2 = edge_index[0].reshape(ech, CHUNK)
    ec2 = edge_index[1].reshape(ech, CHUNK)

    ones_blk = jnp.ones((CHUNK, hh), _F32)
    zero_blk = jnp.zeros((ROWS_PER_TILE, hh), _F32)

    w1b = jnp.zeros((2 * dd, 2 * hh), _F32)
    w1b = w1b.at[0:dd, 0:hh].set(W1).at[dd:2 * dd, hh:2 * hh].set(W1)
    w2b = jnp.zeros((2 * hh, 2 * hh), _F32)
    w2b = w2b.at[0:hh, 0:hh].set(W2).at[hh:2 * hh, hh:2 * hh].set(W2)
    b1p = jnp.concatenate([b1, b1]).reshape(1, 2 * hh)
    b2p = jnp.concatenate([b2, b2]).reshape(1, 2 * hh)
    bi = batch.astype(jnp.int32)
    batch_e = bi[0::2].reshape(1, n // 2)
    batch_o = bi[1::2].reshape(1, n // 2)
    x2 = x.reshape(n // 2, 2 * dd)

    degp = _sc_degree(ec2, padc, ones_blk, zero_blk, ch, lr)
    degv = degp.reshape(NC, NPAD // 2, 128)
    xw = _tc_xw(x2, w1b)
    dis128, y1 = _tc_prelayer(degv, xw)
    z1 = _sc_aggregate(y1.reshape(n, hh), er2, ec2, padr, padc, zero_blk, ch, lr)
    z1v = z1.reshape(NC, NPAD // 2, 128)
    y2 = _tc_midlayer(z1v, y1, dis128, b1p, w2b)
    z2 = _sc_aggregate(y2.reshape(n, hh), er2, ec2, padr, padc, zero_blk, ch, lr)
    z2v = z2.reshape(NC, NPAD // 2, 128)
    return _tc_head(z2v, y2, dis128, b2p, batch_e, batch_o, Wl, bl.reshape(1, 1))
